# baseline (speedup 1.0000x reference)
import functools

import jax
import jax.numpy as jnp
from jax import lax
from jax.experimental import pallas as pl
from jax.experimental.pallas import tpu as pltpu
from jax.experimental.pallas import tpu_sc as plsc

BATCH = 4096
SEQ = 200
D = 64
L = 16
NC = 2
NS = 16
NW = NC * NS
B_PER_W = BATCH // NW
SPLIT = 128
REST = SEQ - SPLIT


def _sc_gather_mean(token_ids, emb_table):
    mesh = plsc.VectorSubcoreMesh(
        core_axis_name="c", subcore_axis_name="s", num_cores=NC, num_subcores=NS
    )

    @functools.partial(
        pl.kernel,
        out_type=jax.ShapeDtypeStruct((BATCH, D), jnp.float32),
        mesh=mesh,
        compiler_params=pltpu.CompilerParams(use_tc_tiling_on_sc=False),
        scratch_types=[
            pltpu.VMEM((B_PER_W, SEQ), jnp.int32),
            pltpu.VMEM((SEQ, D), jnp.float32),
            pltpu.VMEM((B_PER_W, D), jnp.float32),
            pltpu.SemaphoreType.DMA,
        ],
    )
    def k(tok_hbm, table_hbm, out_hbm, idx_v, rows_v, out_v, sem):
        wid = lax.axis_index("s") * NC + lax.axis_index("c")
        base = wid * B_PER_W
        pltpu.sync_copy(tok_hbm.at[pl.ds(base, B_PER_W)], idx_v)

        def row_body(i, carry):
            d1 = pltpu.async_copy(
                table_hbm.at[idx_v.at[i, pl.ds(0, SPLIT)]],
                rows_v.at[pl.ds(0, SPLIT)],
                sem,
            )
            d2 = pltpu.async_copy(
                table_hbm.at[idx_v.at[i, pl.ds(SPLIT, REST)]],
                rows_v.at[pl.ds(SPLIT, REST)],
                sem,
            )
            d1.wait()
            d2.wait()

            def acc_body(j, acc):
                return tuple(
                    acc[d] + rows_v[j, pl.ds(d * L, L)] for d in range(D // L)
                )

            acc = lax.fori_loop(
                0,
                SEQ,
                acc_body,
                tuple(jnp.zeros((L,), jnp.float32) for _ in range(D // L)),
            )
            scale = jnp.float32(1.0 / SEQ)
            for d in range(D // L):
                out_v[i, pl.ds(d * L, L)] = acc[d] * scale
            return carry

        lax.fori_loop(0, B_PER_W, row_body, 0)
        pltpu.sync_copy(out_v, out_hbm.at[pl.ds(base, B_PER_W)])

    return k(token_ids, emb_table)


def _tc_linear(x, wt, b2):
    def mm(x_ref, w_ref, b_ref, o_ref):
        o_ref[...] = (
            jnp.dot(x_ref[...], w_ref[...], preferred_element_type=jnp.float32)
            + b_ref[...]
        )

    return pl.pallas_call(
        mm,
        out_shape=jax.ShapeDtypeStruct((BATCH, D), jnp.float32),
    )(x, wt, b2)


def kernel(token_ids, emb_table, W, b):
    x = _sc_gather_mean(token_ids.astype(jnp.int32), emb_table)
    return _tc_linear(x, W.T, b.reshape(1, D))


# speedup vs baseline: 1.2051x; 1.2051x over previous
import functools

import jax
import jax.numpy as jnp
from jax import lax
from jax.experimental import pallas as pl
from jax.experimental.pallas import tpu as pltpu
from jax.experimental.pallas import tpu_sc as plsc

BATCH = 4096
SEQ = 200
D = 64
L = 16
NC = 2
NS = 16
NW = NC * NS
B_PER_W = BATCH // NW
SPLIT = 128
REST = SEQ - SPLIT


def _sc_gather_mean(token_ids, emb_table):
    mesh = plsc.VectorSubcoreMesh(
        core_axis_name="c", subcore_axis_name="s", num_cores=NC, num_subcores=NS
    )

    NBUF = 4

    @functools.partial(
        pl.kernel,
        out_type=jax.ShapeDtypeStruct((BATCH, D), jnp.float32),
        mesh=mesh,
        compiler_params=pltpu.CompilerParams(use_tc_tiling_on_sc=False),
        scratch_types=[
            pltpu.VMEM((B_PER_W, SEQ), jnp.int32),
            [pltpu.VMEM((SEQ, D), jnp.float32) for _ in range(NBUF)],
            pltpu.VMEM((B_PER_W, D), jnp.float32),
            [pltpu.SemaphoreType.DMA for _ in range(NBUF)],
        ],
    )
    def k(tok_hbm, table_hbm, out_hbm, idx_v, rows, out_v, sems):
        wid = lax.axis_index("s") * NC + lax.axis_index("c")
        base = wid * B_PER_W
        pltpu.sync_copy(tok_hbm.at[pl.ds(base, B_PER_W)], idx_v)

        def start(i, buf, sem):
            pltpu.async_copy(
                table_hbm.at[idx_v.at[i, pl.ds(0, SPLIT)]],
                buf.at[pl.ds(0, SPLIT)],
                sem,
            )
            pltpu.async_copy(
                table_hbm.at[idx_v.at[i, pl.ds(SPLIT, REST)]],
                buf.at[pl.ds(SPLIT, REST)],
                sem,
            )

        def drain(buf, sem):
            pltpu.make_async_copy(table_hbm.at[pl.ds(0, SEQ)], buf, sem).wait()

        for b in range(NBUF):
            start(b, rows[b], sems[b])

        def group_body(g, carry):
            i0 = g * NBUF
            for b in range(NBUF):
                i = i0 + b
                drain(rows[b], sems[b])
                buf = rows[b]

                @plsc.parallel_loop(
                    0,
                    SEQ,
                    unroll=8,
                    carry=tuple(
                        jnp.zeros((L,), jnp.float32) for _ in range(D // L)
                    ),
                )
                def acc(j, c):
                    return tuple(
                        c[d] + buf[j, pl.ds(d * L, L)] for d in range(D // L)
                    )

                scale = jnp.float32(1.0 / SEQ)
                for d in range(D // L):
                    out_v[i, pl.ds(d * L, L)] = acc[d] * scale

                @pl.when(g < B_PER_W // NBUF - 1)
                def _():
                    start(i + NBUF, rows[b], sems[b])

            return carry

        lax.fori_loop(0, B_PER_W // NBUF, group_body, 0)
        pltpu.sync_copy(out_v, out_hbm.at[pl.ds(base, B_PER_W)])

    return k(token_ids, emb_table)


def _tc_linear(x, wt, b2):
    def mm(x_ref, w_ref, b_ref, o_ref):
        o_ref[...] = (
            jnp.dot(x_ref[...], w_ref[...], preferred_element_type=jnp.float32)
            + b_ref[...]
        )

    return pl.pallas_call(
        mm,
        out_shape=jax.ShapeDtypeStruct((BATCH, D), jnp.float32),
    )(x, wt, b2)


def kernel(token_ids, emb_table, W, b):
    x = _sc_gather_mean(token_ids.astype(jnp.int32), emb_table)
    return _tc_linear(x, W.T, b.reshape(1, D))


# speedup vs baseline: 2.1741x; 1.8041x over previous
import functools

import jax
import jax.numpy as jnp
from jax import lax
from jax.experimental import pallas as pl
from jax.experimental.pallas import tpu as pltpu
from jax.experimental.pallas import tpu_sc as plsc

BATCH = 4096
SEQ = 200
D = 64
VOCAB = 1000000
HALF = VOCAB // 2
L = 16
NC = 2
NS = 16
NW = NC * NS
B_PER_W = BATCH // NW
TOK_PER_W = B_PER_W * SEQ
SPLIT = 128
REST = SEQ - SPLIT


BLK = 4096
NBLK = -(-VOCAB // BLK)
VOCAB_PAD = NBLK * BLK


def _tc_transpose(table_t):
    def body(x_ref, o_ref):
        x = x_ref[...]
        z = jnp.concatenate([x[:, : BLK // 2], x[:, BLK // 2 :]], axis=0)
        o_ref[...] = z.T

    return pl.pallas_call(
        body,
        grid=(NBLK,),
        in_specs=[pl.BlockSpec((D, BLK), lambda i: (0, i))],
        out_shape=jax.ShapeDtypeStruct((VOCAB_PAD // 2, 2 * D), jnp.float32),
        out_specs=pl.BlockSpec((BLK // 2, 2 * D), lambda i: (i, 0)),
    )(table_t)


def _sc_gather_mean(token_ids_flat, table_lin):
    mesh = plsc.VectorSubcoreMesh(
        core_axis_name="c", subcore_axis_name="s", num_cores=NC, num_subcores=NS
    )
    NBUF = 4

    @functools.partial(
        pl.kernel,
        out_type=jax.ShapeDtypeStruct((BATCH, D), jnp.float32),
        mesh=mesh,
        compiler_params=pltpu.CompilerParams(use_tc_tiling_on_sc=False),
        scratch_types=[
            pltpu.VMEM((TOK_PER_W,), jnp.int32),
            [pltpu.VMEM((SEQ, D), jnp.float32) for _ in range(NBUF)],
            pltpu.VMEM((B_PER_W, D), jnp.float32),
            [pltpu.SemaphoreType.DMA for _ in range(NBUF)],
        ],
    )
    def k(tok_hbm, table_hbm, out_hbm, idx_v, rows, out_v, sems):
        wid = lax.axis_index("s") * NC + lax.axis_index("c")
        base = wid * B_PER_W
        pltpu.sync_copy(tok_hbm.at[pl.ds(wid * TOK_PER_W, TOK_PER_W)], idx_v)

        @plsc.parallel_loop(0, TOK_PER_W // L, unroll=8)
        def _remap(c):
            v = idx_v[pl.ds(c * L, L)]
            idx_v[pl.ds(c * L, L)] = (
                (v & ~(BLK - 1))
                + ((v & (BLK // 2 - 1)) << 1)
                + ((v & (BLK - 1)) >> 11)
            )

        def start(i, buf, sem):
            pltpu.async_copy(
                table_hbm.at[idx_v.at[pl.ds(i * SEQ, SPLIT)]],
                buf.at[pl.ds(0, SPLIT)],
                sem,
            )
            pltpu.async_copy(
                table_hbm.at[idx_v.at[pl.ds(i * SEQ + SPLIT, REST)]],
                buf.at[pl.ds(SPLIT, REST)],
                sem,
            )

        def drain(buf, sem):
            pltpu.make_async_copy(table_hbm.at[pl.ds(0, SEQ)], buf, sem).wait()

        for b in range(NBUF):
            start(b, rows[b], sems[b])

        def group_body(g, carry):
            i0 = g * NBUF
            for b in range(NBUF):
                i = i0 + b
                drain(rows[b], sems[b])
                buf = rows[b]

                @plsc.parallel_loop(
                    0,
                    SEQ,
                    unroll=8,
                    carry=tuple(
                        jnp.zeros((L,), jnp.float32) for _ in range(D // L)
                    ),
                )
                def acc(j, c):
                    return tuple(
                        c[d] + buf[j, pl.ds(d * L, L)] for d in range(D // L)
                    )

                scale = jnp.float32(1.0 / SEQ)
                for d in range(D // L):
                    out_v[i, pl.ds(d * L, L)] = acc[d] * scale

                @pl.when(g < B_PER_W // NBUF - 1)
                def _():
                    start(i + NBUF, rows[b], sems[b])

            return carry

        lax.fori_loop(0, B_PER_W // NBUF, group_body, 0)
        pltpu.sync_copy(out_v, out_hbm.at[pl.ds(base, B_PER_W)])

    return k(token_ids_flat, table_lin)


def _tc_linear(x, wt, b2):
    def mm(x_ref, w_ref, b_ref, o_ref):
        o_ref[...] = (
            jnp.dot(x_ref[...], w_ref[...], preferred_element_type=jnp.float32)
            + b_ref[...]
        )

    return pl.pallas_call(
        mm,
        out_shape=jax.ShapeDtypeStruct((BATCH, D), jnp.float32),
    )(x, wt, b2)


def kernel(token_ids, emb_table, W, b):
    table_lin = _tc_transpose(emb_table.T).reshape(VOCAB_PAD, D)
    tok_flat = token_ids.astype(jnp.int32).reshape(BATCH * SEQ)
    x = _sc_gather_mean(tok_flat, table_lin)
    return _tc_linear(x, W.T, b.reshape(1, D))


# speedup vs baseline: 2.6624x; 1.2246x over previous
import functools

import jax
import jax.numpy as jnp
from jax import lax
from jax.experimental import pallas as pl
from jax.experimental.pallas import tpu as pltpu
from jax.experimental.pallas import tpu_sc as plsc

BATCH = 4096
SEQ = 200
D = 64
VOCAB = 1000000
HALF = VOCAB // 2
L = 16
NC = 2
NS = 16
NW = NC * NS
B_PER_W = BATCH // NW
TOK_PER_W = B_PER_W * SEQ
SPLIT = 128
REST = SEQ - SPLIT


BLK = 8192
NBLK = -(-VOCAB // BLK)
VOCAB_PAD = NBLK * BLK


def _tc_transpose(table_t):
    def body(x_ref, o_ref):
        x = x_ref[...]
        z = jnp.concatenate([x[:, : BLK // 2], x[:, BLK // 2 :]], axis=0)
        o_ref[...] = z.T

    return pl.pallas_call(
        body,
        grid=(NBLK,),
        in_specs=[pl.BlockSpec((D, BLK), lambda i: (0, i))],
        out_shape=jax.ShapeDtypeStruct((VOCAB_PAD // 2, 2 * D), jnp.float32),
        out_specs=pl.BlockSpec((BLK // 2, 2 * D), lambda i: (i, 0)),
    )(table_t)


def _sc_gather_mean(token_ids_flat, table_lin):
    mesh = plsc.VectorSubcoreMesh(
        core_axis_name="c", subcore_axis_name="s", num_cores=NC, num_subcores=NS
    )
    NBUF = 4

    @functools.partial(
        pl.kernel,
        out_type=jax.ShapeDtypeStruct((BATCH, D), jnp.float32),
        mesh=mesh,
        compiler_params=pltpu.CompilerParams(use_tc_tiling_on_sc=False),
        scratch_types=[
            pltpu.VMEM((TOK_PER_W,), jnp.int32),
            [pltpu.VMEM((SEQ, D), jnp.float32) for _ in range(NBUF)],
            pltpu.VMEM((B_PER_W, D), jnp.float32),
            [pltpu.SemaphoreType.DMA for _ in range(NBUF)],
        ],
    )
    def k(tok_hbm, table_hbm, out_hbm, idx_v, rows, out_v, sems):
        wid = lax.axis_index("s") * NC + lax.axis_index("c")
        base = wid * B_PER_W
        pltpu.sync_copy(tok_hbm.at[pl.ds(wid * TOK_PER_W, TOK_PER_W)], idx_v)

        sh = (BLK // 2).bit_length() - 1

        @plsc.parallel_loop(0, TOK_PER_W // L, unroll=8)
        def _remap(c):
            v = idx_v[pl.ds(c * L, L)]
            idx_v[pl.ds(c * L, L)] = (
                (v & ~(BLK - 1))
                + ((v & (BLK // 2 - 1)) << 1)
                + ((v & (BLK - 1)) >> sh)
            )

        def start(i, buf, sem):
            pltpu.async_copy(
                table_hbm.at[idx_v.at[pl.ds(i * SEQ, SPLIT)]],
                buf.at[pl.ds(0, SPLIT)],
                sem,
            )
            pltpu.async_copy(
                table_hbm.at[idx_v.at[pl.ds(i * SEQ + SPLIT, REST)]],
                buf.at[pl.ds(SPLIT, REST)],
                sem,
            )

        def drain(buf, sem):
            pltpu.make_async_copy(table_hbm.at[pl.ds(0, SEQ)], buf, sem).wait()

        for b in range(NBUF):
            start(b, rows[b], sems[b])

        def group_body(g, carry):
            i0 = g * NBUF
            for b in range(NBUF):
                i = i0 + b
                drain(rows[b], sems[b])
                buf = rows[b]

                @plsc.parallel_loop(
                    0,
                    SEQ,
                    unroll=8,
                    carry=tuple(
                        jnp.zeros((L,), jnp.float32) for _ in range(D // L)
                    ),
                )
                def acc(j, c):
                    return tuple(
                        c[d] + buf[j, pl.ds(d * L, L)] for d in range(D // L)
                    )

                scale = jnp.float32(1.0 / SEQ)
                for d in range(D // L):
                    out_v[i, pl.ds(d * L, L)] = acc[d] * scale

                @pl.when(g < B_PER_W // NBUF - 1)
                def _():
                    start(i + NBUF, rows[b], sems[b])

            return carry

        lax.fori_loop(0, B_PER_W // NBUF, group_body, 0)
        pltpu.sync_copy(out_v, out_hbm.at[pl.ds(base, B_PER_W)])

    return k(token_ids_flat, table_lin)


def _tc_linear(x, wt, b2):
    def mm(x_ref, w_ref, b_ref, o_ref):
        o_ref[...] = (
            jnp.dot(x_ref[...], w_ref[...], preferred_element_type=jnp.float32)
            + b_ref[...]
        )

    return pl.pallas_call(
        mm,
        out_shape=jax.ShapeDtypeStruct((BATCH, D), jnp.float32),
    )(x, wt, b2)


def kernel(token_ids, emb_table, W, b):
    table_lin = _tc_transpose(emb_table.T).reshape(VOCAB_PAD, D)
    tok_flat = token_ids.astype(jnp.int32).reshape(BATCH * SEQ)
    x = _sc_gather_mean(tok_flat, table_lin)
    return _tc_linear(x, W.T, b.reshape(1, D))


# speedup vs baseline: 2.8971x; 1.0881x over previous
import functools

import jax
import jax.numpy as jnp
from jax import lax
from jax.experimental import pallas as pl
from jax.experimental.pallas import tpu as pltpu
from jax.experimental.pallas import tpu_sc as plsc

BATCH = 4096
SEQ = 200
D = 64
VOCAB = 1000000
HALF = VOCAB // 2
L = 16
NC = 2
NS = 16
NW = NC * NS
B_PER_W = BATCH // NW
TOK_PER_W = B_PER_W * SEQ
SPLIT = 128
REST = SEQ - SPLIT


BLK = 16384
NBLK = -(-VOCAB // BLK)
VOCAB_PAD = NBLK * BLK


def _tc_transpose(table_t):
    def body(x_ref, o_ref):
        x = x_ref[...]
        z = jnp.concatenate([x[:, : BLK // 2], x[:, BLK // 2 :]], axis=0)
        o_ref[...] = z.T

    return pl.pallas_call(
        body,
        grid=(NBLK,),
        in_specs=[pl.BlockSpec((D, BLK), lambda i: (0, i))],
        out_shape=jax.ShapeDtypeStruct((VOCAB_PAD // 2, 2 * D), jnp.float32),
        out_specs=pl.BlockSpec((BLK // 2, 2 * D), lambda i: (i, 0)),
    )(table_t)


def _sc_gather_mean(token_ids_flat, table_lin):
    mesh = plsc.VectorSubcoreMesh(
        core_axis_name="c", subcore_axis_name="s", num_cores=NC, num_subcores=NS
    )
    NBUF = 4

    @functools.partial(
        pl.kernel,
        out_type=jax.ShapeDtypeStruct((BATCH, D), jnp.float32),
        mesh=mesh,
        compiler_params=pltpu.CompilerParams(use_tc_tiling_on_sc=False),
        scratch_types=[
            pltpu.VMEM((TOK_PER_W,), jnp.int32),
            [pltpu.VMEM((SEQ, D), jnp.float32) for _ in range(NBUF)],
            pltpu.VMEM((B_PER_W, D), jnp.float32),
            [pltpu.SemaphoreType.DMA for _ in range(NBUF)],
        ],
    )
    def k(tok_hbm, table_hbm, out_hbm, idx_v, rows, out_v, sems):
        wid = lax.axis_index("s") * NC + lax.axis_index("c")
        base = wid * B_PER_W
        pltpu.sync_copy(tok_hbm.at[pl.ds(wid * TOK_PER_W, TOK_PER_W)], idx_v)

        sh = (BLK // 2).bit_length() - 1

        @plsc.parallel_loop(0, TOK_PER_W // L, unroll=8)
        def _remap(c):
            v = idx_v[pl.ds(c * L, L)]
            idx_v[pl.ds(c * L, L)] = (
                (v & ~(BLK - 1))
                + ((v & (BLK // 2 - 1)) << 1)
                + ((v & (BLK - 1)) >> sh)
            )

        def start(i, buf, sem):
            pltpu.async_copy(
                table_hbm.at[idx_v.at[pl.ds(i * SEQ, SPLIT)]],
                buf.at[pl.ds(0, SPLIT)],
                sem,
            )
            pltpu.async_copy(
                table_hbm.at[idx_v.at[pl.ds(i * SEQ + SPLIT, REST)]],
                buf.at[pl.ds(SPLIT, REST)],
                sem,
            )

        def drain(buf, sem):
            pltpu.make_async_copy(table_hbm.at[pl.ds(0, SEQ)], buf, sem).wait()

        for b in range(NBUF):
            start(b, rows[b], sems[b])

        def group_body(g, carry):
            i0 = g * NBUF
            for b in range(NBUF):
                i = i0 + b
                drain(rows[b], sems[b])
                buf = rows[b]

                @plsc.parallel_loop(
                    0,
                    SEQ,
                    unroll=8,
                    carry=tuple(
                        jnp.zeros((L,), jnp.float32) for _ in range(D // L)
                    ),
                )
                def acc(j, c):
                    return tuple(
                        c[d] + buf[j, pl.ds(d * L, L)] for d in range(D // L)
                    )

                scale = jnp.float32(1.0 / SEQ)
                for d in range(D // L):
                    out_v[i, pl.ds(d * L, L)] = acc[d] * scale

                @pl.when(g < B_PER_W // NBUF - 1)
                def _():
                    start(i + NBUF, rows[b], sems[b])

            return carry

        lax.fori_loop(0, B_PER_W // NBUF, group_body, 0)
        pltpu.sync_copy(out_v, out_hbm.at[pl.ds(base, B_PER_W)])

    return k(token_ids_flat, table_lin)


def _tc_linear(x, wt, b2):
    def mm(x_ref, w_ref, b_ref, o_ref):
        o_ref[...] = (
            jnp.dot(x_ref[...], w_ref[...], preferred_element_type=jnp.float32)
            + b_ref[...]
        )

    return pl.pallas_call(
        mm,
        out_shape=jax.ShapeDtypeStruct((BATCH, D), jnp.float32),
    )(x, wt, b2)


def kernel(token_ids, emb_table, W, b):
    table_lin = _tc_transpose(emb_table.T).reshape(VOCAB_PAD, D)
    tok_flat = token_ids.astype(jnp.int32).reshape(BATCH * SEQ)
    x = _sc_gather_mean(tok_flat, table_lin)
    return _tc_linear(x, W.T, b.reshape(1, D))


# speedup vs baseline: 2.9632x; 1.0228x over previous
import functools

import jax
import jax.numpy as jnp
from jax import lax
from jax.experimental import pallas as pl
from jax.experimental.pallas import tpu as pltpu
from jax.experimental.pallas import tpu_sc as plsc

BATCH = 4096
SEQ = 200
D = 64
VOCAB = 1000000
HALF = VOCAB // 2
L = 16
NC = 2
NS = 16
NW = NC * NS
B_PER_W = BATCH // NW
TOK_PER_W = B_PER_W * SEQ
SPLIT = 128
REST = SEQ - SPLIT


BLK = 32768
NBLK = -(-VOCAB // BLK)
VOCAB_PAD = NBLK * BLK


def _tc_transpose(table_t):
    def body(x_ref, o_ref):
        x = x_ref[...]
        z = jnp.concatenate([x[:, : BLK // 2], x[:, BLK // 2 :]], axis=0)
        o_ref[...] = z.T

    return pl.pallas_call(
        body,
        grid=(NBLK,),
        in_specs=[pl.BlockSpec((D, BLK), lambda i: (0, i))],
        out_shape=jax.ShapeDtypeStruct((VOCAB_PAD // 2, 2 * D), jnp.float32),
        out_specs=pl.BlockSpec((BLK // 2, 2 * D), lambda i: (i, 0)),
    )(table_t)


def _sc_gather_mean(token_ids_flat, table_lin):
    mesh = plsc.VectorSubcoreMesh(
        core_axis_name="c", subcore_axis_name="s", num_cores=NC, num_subcores=NS
    )
    NBUF = 4

    @functools.partial(
        pl.kernel,
        out_type=jax.ShapeDtypeStruct((BATCH, D), jnp.float32),
        mesh=mesh,
        compiler_params=pltpu.CompilerParams(use_tc_tiling_on_sc=False),
        scratch_types=[
            pltpu.VMEM((TOK_PER_W,), jnp.int32),
            [pltpu.VMEM((SEQ, D), jnp.float32) for _ in range(NBUF)],
            pltpu.VMEM((B_PER_W, D), jnp.float32),
            [pltpu.SemaphoreType.DMA for _ in range(NBUF)],
        ],
    )
    def k(tok_hbm, table_hbm, out_hbm, idx_v, rows, out_v, sems):
        wid = lax.axis_index("s") * NC + lax.axis_index("c")
        base = wid * B_PER_W
        pltpu.sync_copy(tok_hbm.at[pl.ds(wid * TOK_PER_W, TOK_PER_W)], idx_v)

        sh = (BLK // 2).bit_length() - 1

        @plsc.parallel_loop(0, TOK_PER_W // L, unroll=8)
        def _remap(c):
            v = idx_v[pl.ds(c * L, L)]
            idx_v[pl.ds(c * L, L)] = (
                (v & ~(BLK - 1))
                + ((v & (BLK // 2 - 1)) << 1)
                + ((v & (BLK - 1)) >> sh)
            )

        def start(i, buf, sem):
            pltpu.async_copy(
                table_hbm.at[idx_v.at[pl.ds(i * SEQ, SPLIT)]],
                buf.at[pl.ds(0, SPLIT)],
                sem,
            )
            pltpu.async_copy(
                table_hbm.at[idx_v.at[pl.ds(i * SEQ + SPLIT, REST)]],
                buf.at[pl.ds(SPLIT, REST)],
                sem,
            )

        def drain(buf, sem):
            pltpu.make_async_copy(table_hbm.at[pl.ds(0, SEQ)], buf, sem).wait()

        for b in range(NBUF):
            start(b, rows[b], sems[b])

        def group_body(g, carry):
            i0 = g * NBUF
            for b in range(NBUF):
                i = i0 + b
                drain(rows[b], sems[b])
                buf = rows[b]

                @plsc.parallel_loop(
                    0,
                    SEQ,
                    unroll=8,
                    carry=tuple(
                        jnp.zeros((L,), jnp.float32) for _ in range(D // L)
                    ),
                )
                def acc(j, c):
                    return tuple(
                        c[d] + buf[j, pl.ds(d * L, L)] for d in range(D // L)
                    )

                scale = jnp.float32(1.0 / SEQ)
                for d in range(D // L):
                    out_v[i, pl.ds(d * L, L)] = acc[d] * scale

                @pl.when(g < B_PER_W // NBUF - 1)
                def _():
                    start(i + NBUF, rows[b], sems[b])

            return carry

        lax.fori_loop(0, B_PER_W // NBUF, group_body, 0)
        pltpu.sync_copy(out_v, out_hbm.at[pl.ds(base, B_PER_W)])

    return k(token_ids_flat, table_lin)


def _tc_linear(x, wt, b2):
    def mm(x_ref, w_ref, b_ref, o_ref):
        o_ref[...] = (
            jnp.dot(x_ref[...], w_ref[...], preferred_element_type=jnp.float32)
            + b_ref[...]
        )

    return pl.pallas_call(
        mm,
        out_shape=jax.ShapeDtypeStruct((BATCH, D), jnp.float32),
    )(x, wt, b2)


def kernel(token_ids, emb_table, W, b):
    table_lin = _tc_transpose(emb_table.T).reshape(VOCAB_PAD, D)
    tok_flat = token_ids.astype(jnp.int32).reshape(BATCH * SEQ)
    x = _sc_gather_mean(tok_flat, table_lin)
    return _tc_linear(x, W.T, b.reshape(1, D))
